# baseline (speedup 1.0000x reference)
import jax
import jax.numpy as jnp
from jax import lax
from jax.experimental import pallas as pl
from jax.experimental.pallas import tpu as pltpu, tpu_sc as plsc

N_NODES = 100000
N_EDGES = 3200000
DELTA_T = 0.01
MU = 0.01

LANES = 16
ROW = 128
N_ROWS = N_EDGES // ROW
C_ROWS = 8
CHUNK = C_ROWS * ROW
BIG_TILES = 21
N_MAIN = 96
N_GRP = N_MAIN // 3

NODE_SLICE = 6256
LAST_SLICE = N_NODES - 15 * NODE_SLICE
CB = 2048


def _edge_pass(ei_hbm, cat_hbm, g_hbm, p0_out, p1_out,
               u_v, cb_v, src_v0, src_v1, src_v2, dst_v0, dst_v1, dst_v2,
               dst2_v0, dst2_v1, dst2_v2, w_v0, w_v1, w_v2,
               msg_v0, msg_v1, msg_v2,
               acc, st_sem0, st_sem1, st_sem2, sc_sem0, sc_sem1, sc_sem2):
    src_v = (src_v0, src_v1, src_v2)
    dst_v = (dst_v0, dst_v1, dst_v2)
    dst2_v = (dst2_v0, dst2_v1, dst2_v2)
    w_v = (w_v0, w_v1, w_v2)
    msg_v = (msg_v0, msg_v1, msg_v2)
    st_sem = (st_sem0, st_sem1, st_sem2)
    sc_sem = (sc_sem0, sc_sem1, sc_sem2)

    c = lax.axis_index("c")
    s = lax.axis_index("s")
    wid = c * 16 + s

    pltpu.sync_copy(g_hbm, u_v)

    def zero16(i, _):
        cb_v[pl.ds(i * LANES, LANES)] = jnp.zeros((LANES,), jnp.float32)
        return 0

    lax.fori_loop(0, CB // LANES, zero16, 0)
    for q in range(3):
        pltpu.sync_copy(cb_v, acc.at[pl.ds(s * NODE_SLICE + q * CB, CB)])

    @pl.when(s < 15)
    def _():
        pltpu.sync_copy(cb_v.at[pl.ds(0, NODE_SLICE - 3 * CB)],
                        acc.at[pl.ds(s * NODE_SLICE + 3 * CB,
                                     NODE_SLICE - 3 * CB)])

    @pl.when(s == 15)
    def _():
        pltpu.sync_copy(cb_v.at[pl.ds(0, LAST_SLICE - 3 * CB)],
                        acc.at[pl.ds(15 * NODE_SLICE + 3 * CB,
                                     LAST_SLICE - 3 * CB)])

    plsc.subcore_barrier()

    is_big = wid < BIG_TILES
    r0 = 784 * wid - C_ROWS * jnp.maximum(wid - BIG_TILES, 0)

    def fire_stage(b, chunk_idx):
        e0 = (r0 + chunk_idx * C_ROWS) * ROW
        pltpu.async_copy(ei_hbm.at[0, pl.ds(e0, CHUNK)], src_v[b], st_sem[b])
        pltpu.async_copy(cat_hbm.at[pl.ds(e0, CHUNK)], dst_v[b], st_sem[b])
        pltpu.async_copy(cat_hbm.at[pl.ds(N_EDGES + e0, CHUNK)], w_v[b],
                         st_sem[b])

    def wait_stage(b):
        pltpu.make_async_copy(ei_hbm.at[0, pl.ds(0, CHUNK)], src_v[b],
                              st_sem[b]).wait()
        pltpu.make_async_copy(cat_hbm.at[pl.ds(0, CHUNK)], dst_v[b],
                              st_sem[b]).wait()
        pltpu.make_async_copy(cat_hbm.at[pl.ds(0, CHUNK)], w_v[b],
                              st_sem[b]).wait()

    def compute_rows(b):
        @plsc.parallel_loop(0, CHUNK // LANES, unroll=8)
        def _(i, b=b):
            off = i * LANES
            j = i // (ROW // LANES)
            k = i % (ROW // LANES)
            s_idx = src_v[b][pl.ds(off, LANES)]
            d_idx = dst_v[b][pl.ds(off, LANES)]
            wv = plsc.bitcast(w_v[b][pl.ds(off, LANES)], jnp.float32)
            us = plsc.load_gather(u_v, [s_idx])
            ud = plsc.load_gather(u_v, [d_idx])
            msg_v[b][j, pl.ds(k * LANES, LANES)] = (us - ud) * wv
            dst2_v[b][j, pl.ds(k * LANES, LANES)] = d_idx

    def fire_scatter(b):
        for j in range(C_ROWS):
            pltpu.async_copy(msg_v[b].at[j], acc.at[dst2_v[b].at[j]],
                             sc_sem[b], add=True)

    def drain_scatter(b):
        for j in range(C_ROWS):
            pltpu.make_async_copy(msg_v[b].at[j], acc.at[dst2_v[b].at[j]],
                                  sc_sem[b]).wait()

    fire_stage(0, 0)
    fire_stage(1, 1)

    def pipe_grp(g, _):
        for b in range(3):
            i = 3 * g + b
            wait_stage(b)
            compute_rows(b)
            fire_scatter(b)
            if b == 0:
                @pl.when(g > 0)
                def _():
                    drain_scatter(2)
                fire_stage(2, i + 2)
            else:
                bn = (b + 2) % 3
                drain_scatter(bn)

                @pl.when(g < N_GRP - 1)
                def _(bn=bn, i=i):
                    fire_stage(bn, i + 2)
        return 0

    lax.fori_loop(0, N_GRP, pipe_grp, 0)

    drain_scatter(2)

    fire_stage(0, N_MAIN)
    wait_stage(0)
    compute_rows(0)
    fire_scatter(0)

    @pl.when(is_big)
    def _():
        fire_stage(1, N_MAIN + 1)
        wait_stage(1)
        compute_rows(1)
        fire_scatter(1)
        drain_scatter(1)

    drain_scatter(0)

    plsc.subcore_barrier()

    p_out = (p0_out, p1_out)

    def wb_core(dst_hbm_ref):
        for q in range(3):
            off = s * NODE_SLICE + q * CB
            pltpu.sync_copy(acc.at[pl.ds(off, CB)], cb_v)
            pltpu.sync_copy(cb_v, dst_hbm_ref.at[pl.ds(off, CB)])

        @pl.when(s < 15)
        def _():
            n = NODE_SLICE - 3 * CB
            off = s * NODE_SLICE + 3 * CB
            pltpu.sync_copy(acc.at[pl.ds(off, n)], cb_v.at[pl.ds(0, n)])
            pltpu.sync_copy(cb_v.at[pl.ds(0, n)],
                            dst_hbm_ref.at[pl.ds(off, n)])

        @pl.when(s == 15)
        def _():
            n = LAST_SLICE - 3 * CB
            off = 15 * NODE_SLICE + 3 * CB
            pltpu.sync_copy(acc.at[pl.ds(off, n)], cb_v.at[pl.ds(0, n)])
            pltpu.sync_copy(cb_v.at[pl.ds(0, n)],
                            dst_hbm_ref.at[pl.ds(off, n)])

    @pl.when(c == 0)
    def _():
        wb_core(p0_out)

    @pl.when(c == 1)
    def _():
        wb_core(p1_out)


def _make_pass():
    mesh = plsc.VectorSubcoreMesh(core_axis_name="c", subcore_axis_name="s")
    node = jax.ShapeDtypeStruct((N_NODES,), jnp.float32)
    scratches = (
        [pltpu.VMEM((N_NODES,), jnp.float32),
         pltpu.VMEM((CB,), jnp.float32)]
        + [pltpu.VMEM((CHUNK,), jnp.int32)] * 3
        + [pltpu.VMEM((CHUNK,), jnp.int32)] * 3
        + [pltpu.VMEM((C_ROWS, ROW), jnp.int32)] * 3
        + [pltpu.VMEM((CHUNK,), jnp.int32)] * 3
        + [pltpu.VMEM((C_ROWS, ROW), jnp.float32)] * 3
        + [pltpu.VMEM_SHARED((N_NODES,), jnp.float32)]
        + [pltpu.SemaphoreType.DMA] * 6
    )
    return pl.kernel(
        _edge_pass,
        out_type=(node, node),
        mesh=mesh,
        scratch_types=scratches,
        compiler_params=pltpu.CompilerParams(needs_layout_passes=False),
        name="burger_pass",
    )


def _sum_body(p0_ref, p1_ref, o_ref):
    o_ref[...] = p0_ref[...] + p1_ref[...]


def _residual_body(ut_ref, ut1_ref, s1_ref, p0_ref, p1_ref, o_ref):
    ut = ut_ref[...]
    ut1 = ut1_ref[...]
    s1 = s1_ref[...]
    s2 = p0_ref[...] + p1_ref[...]
    o_ref[...] = (ut - ut1) / DELTA_T + s1 * ut1 - MU * s2


def kernel(x_t, x_t1, edge_index, edge_attr):
    u_t = x_t[:, 0]
    u_t1 = x_t1[:, 0]
    cat = jnp.concatenate(
        [edge_index[1],
         jax.lax.bitcast_convert_type(edge_attr[:, 0], jnp.int32)])

    edge_pass = _make_pass()
    shape2d = (8, N_NODES // 8)

    p0, p1 = edge_pass(edge_index, cat, u_t1)
    s1 = pl.pallas_call(
        _sum_body,
        out_shape=jax.ShapeDtypeStruct(shape2d, jnp.float32),
    )(p0.reshape(shape2d), p1.reshape(shape2d)).reshape(N_NODES)

    q0, q1 = edge_pass(edge_index, cat, s1)
    loss = pl.pallas_call(
        _residual_body,
        out_shape=jax.ShapeDtypeStruct(shape2d, jnp.float32),
    )(u_t.reshape(shape2d), u_t1.reshape(shape2d), s1.reshape(shape2d),
      q0.reshape(shape2d), q1.reshape(shape2d))
    return loss.reshape(N_NODES)


# speedup vs baseline: 1.1171x; 1.1171x over previous
import jax
import jax.numpy as jnp
from jax import lax
from jax.experimental import pallas as pl
from jax.experimental.pallas import tpu as pltpu, tpu_sc as plsc

N_NODES = 100000
N_EDGES = 3200000
DELTA_T = 0.01
MU = 0.01

LANES = 16
ROW = 128
N_ROWS = N_EDGES // ROW
C_ROWS = 8
CHUNK = C_ROWS * ROW
BIG_TILES = 21
N_MAIN = 96
N_GRP = N_MAIN // 3

NODE_SLICE = 6256
LAST_SLICE = N_NODES - 15 * NODE_SLICE
CB = 2048


def _edge_pass(ei_hbm, dst_hbm, w_hbm, g_hbm, p0_out, p1_out,
               u_v, cb_v, src_v0, src_v1, src_v2, dst_v0, dst_v1, dst_v2,
               dst2_v0, dst2_v1, dst2_v2, w_v0, w_v1, w_v2,
               msg_v0, msg_v1, msg_v2,
               acc, st_sem0, st_sem1, st_sem2, sc_sem0, sc_sem1, sc_sem2):
    src_v = (src_v0, src_v1, src_v2)
    dst_v = (dst_v0, dst_v1, dst_v2)
    dst2_v = (dst2_v0, dst2_v1, dst2_v2)
    w_v = (w_v0, w_v1, w_v2)
    msg_v = (msg_v0, msg_v1, msg_v2)
    st_sem = (st_sem0, st_sem1, st_sem2)
    sc_sem = (sc_sem0, sc_sem1, sc_sem2)

    c = lax.axis_index("c")
    s = lax.axis_index("s")
    wid = c * 16 + s

    pltpu.sync_copy(g_hbm, u_v)

    def zero16(i, _):
        cb_v[pl.ds(i * LANES, LANES)] = jnp.zeros((LANES,), jnp.float32)
        return 0

    lax.fori_loop(0, CB // LANES, zero16, 0)
    for q in range(3):
        pltpu.sync_copy(cb_v, acc.at[pl.ds(s * NODE_SLICE + q * CB, CB)])

    @pl.when(s < 15)
    def _():
        pltpu.sync_copy(cb_v.at[pl.ds(0, NODE_SLICE - 3 * CB)],
                        acc.at[pl.ds(s * NODE_SLICE + 3 * CB,
                                     NODE_SLICE - 3 * CB)])

    @pl.when(s == 15)
    def _():
        pltpu.sync_copy(cb_v.at[pl.ds(0, LAST_SLICE - 3 * CB)],
                        acc.at[pl.ds(15 * NODE_SLICE + 3 * CB,
                                     LAST_SLICE - 3 * CB)])

    plsc.subcore_barrier()

    is_big = wid < BIG_TILES
    r0 = 784 * wid - C_ROWS * jnp.maximum(wid - BIG_TILES, 0)

    def fire_stage(b, chunk_idx):
        e0 = (r0 + chunk_idx * C_ROWS) * ROW
        pltpu.async_copy(ei_hbm.at[0, pl.ds(e0, CHUNK)], src_v[b], st_sem[b])
        pltpu.async_copy(dst_hbm.at[pl.ds(e0, CHUNK)], dst_v[b], st_sem[b])
        pltpu.async_copy(w_hbm.at[pl.ds(e0, CHUNK)], w_v[b], st_sem[b])

    def wait_stage(b):
        pltpu.make_async_copy(ei_hbm.at[0, pl.ds(0, CHUNK)], src_v[b],
                              st_sem[b]).wait()
        pltpu.make_async_copy(dst_hbm.at[pl.ds(0, CHUNK)], dst_v[b],
                              st_sem[b]).wait()
        pltpu.make_async_copy(w_hbm.at[pl.ds(0, CHUNK)], w_v[b],
                              st_sem[b]).wait()

    def compute_rows(b):
        @plsc.parallel_loop(0, CHUNK // LANES, unroll=8)
        def _(i, b=b):
            off = i * LANES
            j = i // (ROW // LANES)
            k = i % (ROW // LANES)
            s_idx = src_v[b][pl.ds(off, LANES)]
            d_idx = dst_v[b][pl.ds(off, LANES)]
            wv = w_v[b][pl.ds(off, LANES)]
            us = plsc.load_gather(u_v, [s_idx])
            ud = plsc.load_gather(u_v, [d_idx])
            msg_v[b][j, pl.ds(k * LANES, LANES)] = (us - ud) * wv
            dst2_v[b][j, pl.ds(k * LANES, LANES)] = d_idx

    def fire_scatter(b):
        for j in range(C_ROWS):
            pltpu.async_copy(msg_v[b].at[j], acc.at[dst2_v[b].at[j]],
                             sc_sem[b], add=True)

    def drain_scatter(b):
        for j in range(C_ROWS):
            pltpu.make_async_copy(msg_v[b].at[j], acc.at[dst2_v[b].at[j]],
                                  sc_sem[b]).wait()

    fire_stage(0, 0)
    fire_stage(1, 1)

    def pipe_grp(g, _):
        for b in range(3):
            i = 3 * g + b
            wait_stage(b)
            compute_rows(b)
            fire_scatter(b)
            if b == 0:
                @pl.when(g > 0)
                def _():
                    drain_scatter(2)
                fire_stage(2, i + 2)
            else:
                bn = (b + 2) % 3
                drain_scatter(bn)

                @pl.when(g < N_GRP - 1)
                def _(bn=bn, i=i):
                    fire_stage(bn, i + 2)
        return 0

    lax.fori_loop(0, N_GRP, pipe_grp, 0)

    drain_scatter(2)

    fire_stage(0, N_MAIN)
    wait_stage(0)
    compute_rows(0)
    fire_scatter(0)

    @pl.when(is_big)
    def _():
        fire_stage(1, N_MAIN + 1)
        wait_stage(1)
        compute_rows(1)
        fire_scatter(1)
        drain_scatter(1)

    drain_scatter(0)

    plsc.subcore_barrier()

    p_out = (p0_out, p1_out)

    def wb_core(dst_hbm_ref):
        for q in range(3):
            off = s * NODE_SLICE + q * CB
            pltpu.sync_copy(acc.at[pl.ds(off, CB)], cb_v)
            pltpu.sync_copy(cb_v, dst_hbm_ref.at[pl.ds(off, CB)])

        @pl.when(s < 15)
        def _():
            n = NODE_SLICE - 3 * CB
            off = s * NODE_SLICE + 3 * CB
            pltpu.sync_copy(acc.at[pl.ds(off, n)], cb_v.at[pl.ds(0, n)])
            pltpu.sync_copy(cb_v.at[pl.ds(0, n)],
                            dst_hbm_ref.at[pl.ds(off, n)])

        @pl.when(s == 15)
        def _():
            n = LAST_SLICE - 3 * CB
            off = 15 * NODE_SLICE + 3 * CB
            pltpu.sync_copy(acc.at[pl.ds(off, n)], cb_v.at[pl.ds(0, n)])
            pltpu.sync_copy(cb_v.at[pl.ds(0, n)],
                            dst_hbm_ref.at[pl.ds(off, n)])

    @pl.when(c == 0)
    def _():
        wb_core(p0_out)

    @pl.when(c == 1)
    def _():
        wb_core(p1_out)


def _make_pass():
    mesh = plsc.VectorSubcoreMesh(core_axis_name="c", subcore_axis_name="s")
    node = jax.ShapeDtypeStruct((N_NODES,), jnp.float32)
    scratches = (
        [pltpu.VMEM((N_NODES,), jnp.float32),
         pltpu.VMEM((CB,), jnp.float32)]
        + [pltpu.VMEM((CHUNK,), jnp.int32)] * 3
        + [pltpu.VMEM((CHUNK,), jnp.int32)] * 3
        + [pltpu.VMEM((C_ROWS, ROW), jnp.int32)] * 3
        + [pltpu.VMEM((CHUNK,), jnp.float32)] * 3
        + [pltpu.VMEM((C_ROWS, ROW), jnp.float32)] * 3
        + [pltpu.VMEM_SHARED((N_NODES,), jnp.float32)]
        + [pltpu.SemaphoreType.DMA] * 6
    )
    return pl.kernel(
        _edge_pass,
        out_type=(node, node),
        mesh=mesh,
        scratch_types=scratches,
        compiler_params=pltpu.CompilerParams(needs_layout_passes=False),
        name="burger_pass",
    )


def _sum_body(p0_ref, p1_ref, o_ref):
    o_ref[...] = p0_ref[...] + p1_ref[...]


def _residual_body(ut_ref, ut1_ref, s1_ref, p0_ref, p1_ref, o_ref):
    ut = ut_ref[...]
    ut1 = ut1_ref[...]
    s1 = s1_ref[...]
    s2 = p0_ref[...] + p1_ref[...]
    o_ref[...] = (ut - ut1) / DELTA_T + s1 * ut1 - MU * s2


def kernel(x_t, x_t1, edge_index, edge_attr):
    u_t = x_t[:, 0]
    u_t1 = x_t1[:, 0]
    dst1 = edge_index[1]
    w1 = edge_attr[:, 0]

    edge_pass = _make_pass()
    shape2d = (8, N_NODES // 8)

    p0, p1 = edge_pass(edge_index, dst1, w1, u_t1)
    s1 = pl.pallas_call(
        _sum_body,
        out_shape=jax.ShapeDtypeStruct(shape2d, jnp.float32),
    )(p0.reshape(shape2d), p1.reshape(shape2d)).reshape(N_NODES)

    q0, q1 = edge_pass(edge_index, dst1, w1, s1)
    loss = pl.pallas_call(
        _residual_body,
        out_shape=jax.ShapeDtypeStruct(shape2d, jnp.float32),
    )(u_t.reshape(shape2d), u_t1.reshape(shape2d), s1.reshape(shape2d),
      q0.reshape(shape2d), q1.reshape(shape2d))
    return loss.reshape(N_NODES)
